# baseline (device time: 124209 ns/iter reference)
import jax
import jax.numpy as jnp
from jax import lax
from jax.experimental import pallas as pl
from jax.experimental.pallas import tpu as pltpu

N_DEV = 8


def kernel(x, router_W, route_idx, expert_W):
    n_tok, d_model = x.shape
    n_local, _, d_hidden = expert_W.shape
    rows_per = n_tok // N_DEV
    n_experts = router_W.shape[1]
    n_steps = N_DEV - 1

    def body(x_ref, rw_ref, idx_ref, ew_ref, out_ref,
             partial_ref, send_ref, recv_ref, send_sems, recv_sems):
        my = lax.axis_index("i")
        left = lax.rem(my + N_DEV - 1, N_DEV)
        right = lax.rem(my + 1, N_DEV)

        xf = x_ref[:, :]
        scores = jnp.dot(xf, rw_ref[:, :], preferred_element_type=jnp.float32)
        m = jnp.max(scores, axis=1, keepdims=True)
        p = jnp.exp(scores - m)
        denom = jnp.sum(p, axis=1, keepdims=True)
        probs = p / denom

        cols = lax.broadcasted_iota(jnp.int32, (n_tok, n_experts), 1)
        idx0 = idx_ref[:, 0:1]
        idx1 = idx_ref[:, 1:2]
        g0 = jnp.sum(jnp.where(cols == idx0, probs, 0.0), axis=1, keepdims=True)
        g1 = jnp.sum(jnp.where(cols == idx1, probs, 0.0), axis=1, keepdims=True)
        gs = g0 + g1
        w0 = g0 / gs
        w1 = g1 / gs

        acc = jnp.zeros((n_tok, d_hidden), dtype=jnp.float32)
        for e in range(n_local):
            ge = my * n_local + e
            wt = (jnp.where(idx0 == ge, w0, 0.0)
                  + jnp.where(idx1 == ge, w1, 0.0))
            xs = (xf * wt).astype(jnp.bfloat16)
            we = ew_ref[e].astype(jnp.bfloat16)
            acc = acc + jnp.dot(xs, we, preferred_element_type=jnp.float32)
        partial_ref[:, :] = acc

        barrier_sem = pltpu.get_barrier_semaphore()
        for nbr in (left, right):
            pl.semaphore_signal(
                barrier_sem, inc=1,
                device_id=(nbr,), device_id_type=pl.DeviceIdType.MESH,
            )
        pl.semaphore_wait(barrier_sem, 2)

        for s in range(n_steps):
            send_idx = lax.rem(my + (n_steps - s), N_DEV)
            recv_idx = lax.rem(my + (n_steps - s - 1), N_DEV)
            if s == 0:
                send_ref[:, :] = partial_ref[pl.ds(send_idx * rows_per, rows_per), :]
            rdma = pltpu.make_async_remote_copy(
                src_ref=send_ref,
                dst_ref=recv_ref.at[s],
                send_sem=send_sems.at[s],
                recv_sem=recv_sems.at[s],
                device_id=(right,),
                device_id_type=pl.DeviceIdType.MESH,
            )
            rdma.start()
            rdma.wait()
            combined = (recv_ref[s]
                        + partial_ref[pl.ds(recv_idx * rows_per, rows_per), :])
            if s < n_steps - 1:
                send_ref[:, :] = combined
            else:
                out_ref[:, :] = combined

    return pl.pallas_call(
        body,
        out_shape=jax.ShapeDtypeStruct((rows_per, d_hidden), jnp.float32),
        in_specs=[
            pl.BlockSpec(memory_space=pltpu.VMEM),
            pl.BlockSpec(memory_space=pltpu.VMEM),
            pl.BlockSpec(memory_space=pltpu.VMEM),
            pl.BlockSpec(memory_space=pltpu.VMEM),
        ],
        out_specs=pl.BlockSpec(memory_space=pltpu.VMEM),
        scratch_shapes=[
            pltpu.VMEM((n_tok, d_hidden), jnp.float32),
            pltpu.VMEM((rows_per, d_hidden), jnp.float32),
            pltpu.VMEM((n_steps, rows_per, d_hidden), jnp.float32),
            pltpu.SemaphoreType.DMA((n_steps,)),
            pltpu.SemaphoreType.DMA((n_steps,)),
        ],
        compiler_params=pltpu.CompilerParams(collective_id=0),
    )(x, router_W, route_idx, expert_W)


# device time: 84681 ns/iter; 1.4668x vs baseline; 1.4668x over previous
import jax
import jax.numpy as jnp
from jax import lax
from jax.experimental import pallas as pl
from jax.experimental.pallas import tpu as pltpu

N_DEV = 8


def kernel(x, router_W, route_idx, expert_W):
    n_tok, d_model = x.shape
    n_local, _, d_hidden = expert_W.shape
    rows_per = n_tok // N_DEV
    n_experts = router_W.shape[1]
    n_steps = N_DEV - 1

    def body(x_ref, rw_ref, idx_ref, ew_ref, out_ref,
             partial_ref, send_ref, recv_ref, send_sems, recv_sems):
        my = lax.axis_index("i")
        left = lax.rem(my + N_DEV - 1, N_DEV)
        right = lax.rem(my + 1, N_DEV)

        xf = x_ref[:, :]
        scores = jnp.dot(xf, rw_ref[:, :], preferred_element_type=jnp.float32)
        m = jnp.max(scores, axis=1, keepdims=True)
        p = jnp.exp(scores - m)
        denom = jnp.sum(p, axis=1, keepdims=True)
        probs = p / denom

        cols = lax.broadcasted_iota(jnp.int32, (n_tok, n_experts), 1)
        idx0 = idx_ref[:, 0:1]
        idx1 = idx_ref[:, 1:2]
        g0 = jnp.sum(jnp.where(cols == idx0, probs, 0.0), axis=1, keepdims=True)
        g1 = jnp.sum(jnp.where(cols == idx1, probs, 0.0), axis=1, keepdims=True)
        gs = g0 + g1
        w0 = g0 / gs
        w1 = g1 / gs

        acc = jnp.zeros((n_tok, d_hidden), dtype=jnp.float32)
        for e in range(n_local):
            ge = my * n_local + e
            wt = (jnp.where(idx0 == ge, w0, 0.0)
                  + jnp.where(idx1 == ge, w1, 0.0))
            xs = (xf * wt).astype(jnp.bfloat16)
            we = ew_ref[e].astype(jnp.bfloat16)
            acc = acc + jnp.dot(xs, we, preferred_element_type=jnp.float32)
        partial_ref[:, :] = acc

        barrier_sem = pltpu.get_barrier_semaphore()
        for nbr in (left, right):
            pl.semaphore_signal(
                barrier_sem, inc=1,
                device_id=(nbr,), device_id_type=pl.DeviceIdType.MESH,
            )
        pl.semaphore_wait(barrier_sem, 2)

        for s in range(n_steps):
            send_idx = lax.rem(my + (n_steps - s), N_DEV)
            recv_idx = lax.rem(my + (n_steps - s - 1), N_DEV)
            if s == 0:
                send_ref[:, :] = partial_ref[
                    pl.ds(send_idx * rows_per, rows_per), :
                ].astype(jnp.bfloat16)
            rdma = pltpu.make_async_remote_copy(
                src_ref=send_ref,
                dst_ref=recv_ref.at[s],
                send_sem=send_sems.at[s],
                recv_sem=recv_sems.at[s],
                device_id=(right,),
                device_id_type=pl.DeviceIdType.MESH,
            )
            rdma.start()
            rdma.wait()
            combined = (recv_ref[s].astype(jnp.float32)
                        + partial_ref[pl.ds(recv_idx * rows_per, rows_per), :])
            if s < n_steps - 1:
                send_ref[:, :] = combined.astype(jnp.bfloat16)
            else:
                out_ref[:, :] = combined

    return pl.pallas_call(
        body,
        out_shape=jax.ShapeDtypeStruct((rows_per, d_hidden), jnp.float32),
        in_specs=[
            pl.BlockSpec(memory_space=pltpu.VMEM),
            pl.BlockSpec(memory_space=pltpu.VMEM),
            pl.BlockSpec(memory_space=pltpu.VMEM),
            pl.BlockSpec(memory_space=pltpu.VMEM),
        ],
        out_specs=pl.BlockSpec(memory_space=pltpu.VMEM),
        scratch_shapes=[
            pltpu.VMEM((n_tok, d_hidden), jnp.float32),
            pltpu.VMEM((rows_per, d_hidden), jnp.bfloat16),
            pltpu.VMEM((n_steps, rows_per, d_hidden), jnp.bfloat16),
            pltpu.SemaphoreType.DMA((n_steps,)),
            pltpu.SemaphoreType.DMA((n_steps,)),
        ],
        compiler_params=pltpu.CompilerParams(collective_id=0),
    )(x, router_W, route_idx, expert_W)


# device time: 68066 ns/iter; 1.8248x vs baseline; 1.2441x over previous
import jax
import jax.numpy as jnp
from jax import lax
from jax.experimental import pallas as pl
from jax.experimental.pallas import tpu as pltpu

N_DEV = 8


def kernel(x, router_W, route_idx, expert_W):
    n_tok, d_model = x.shape
    n_local, _, d_hidden = expert_W.shape
    rows_per = n_tok // N_DEV
    n_experts = router_W.shape[1]
    n_steps = N_DEV - 1

    half = rows_per // 2

    def body(x_ref, rw_ref, idx_ref, ew_ref, out_ref,
             partial_ref, send_cw_ref, send_ccw_ref, recv_cw_ref, recv_ccw_ref,
             send_cw_sems, recv_cw_sems, send_ccw_sems, recv_ccw_sems):
        my = lax.axis_index("i")
        left = lax.rem(my + N_DEV - 1, N_DEV)
        right = lax.rem(my + 1, N_DEV)

        xf = x_ref[:, :]
        scores = jnp.dot(xf, rw_ref[:, :], preferred_element_type=jnp.float32)
        m = jnp.max(scores, axis=1, keepdims=True)
        p = jnp.exp(scores - m)
        denom = jnp.sum(p, axis=1, keepdims=True)
        probs = p / denom

        cols = lax.broadcasted_iota(jnp.int32, (n_tok, n_experts), 1)
        idx0 = idx_ref[:, 0:1]
        idx1 = idx_ref[:, 1:2]
        g0 = jnp.sum(jnp.where(cols == idx0, probs, 0.0), axis=1, keepdims=True)
        g1 = jnp.sum(jnp.where(cols == idx1, probs, 0.0), axis=1, keepdims=True)
        gs = g0 + g1
        w0 = g0 / gs
        w1 = g1 / gs

        acc = jnp.zeros((n_tok, d_hidden), dtype=jnp.float32)
        for e in range(n_local):
            ge = my * n_local + e
            wt = (jnp.where(idx0 == ge, w0, 0.0)
                  + jnp.where(idx1 == ge, w1, 0.0))
            xs = (xf * wt).astype(jnp.bfloat16)
            we = ew_ref[e].astype(jnp.bfloat16)
            acc = acc + jnp.dot(xs, we, preferred_element_type=jnp.float32)
        partial_ref[:, :] = acc

        barrier_sem = pltpu.get_barrier_semaphore()
        for nbr in (left, right):
            pl.semaphore_signal(
                barrier_sem, inc=1,
                device_id=(nbr,), device_id_type=pl.DeviceIdType.MESH,
            )
        pl.semaphore_wait(barrier_sem, 2)

        for s in range(n_steps):
            cw_send_idx = lax.rem(my + (n_steps - s), N_DEV)
            cw_recv_idx = lax.rem(my + (n_steps - s - 1), N_DEV)
            ccw_send_idx = lax.rem(my + s + 1, N_DEV)
            ccw_recv_idx = lax.rem(my + s + 2, N_DEV)
            if s == 0:
                send_cw_ref[:, :] = partial_ref[
                    pl.ds(cw_send_idx * rows_per, half), :
                ].astype(jnp.bfloat16)
                send_ccw_ref[:, :] = partial_ref[
                    pl.ds(ccw_send_idx * rows_per + half, half), :
                ].astype(jnp.bfloat16)
            rdma_cw = pltpu.make_async_remote_copy(
                src_ref=send_cw_ref,
                dst_ref=recv_cw_ref.at[s],
                send_sem=send_cw_sems.at[s],
                recv_sem=recv_cw_sems.at[s],
                device_id=(right,),
                device_id_type=pl.DeviceIdType.MESH,
            )
            rdma_ccw = pltpu.make_async_remote_copy(
                src_ref=send_ccw_ref,
                dst_ref=recv_ccw_ref.at[s],
                send_sem=send_ccw_sems.at[s],
                recv_sem=recv_ccw_sems.at[s],
                device_id=(left,),
                device_id_type=pl.DeviceIdType.MESH,
            )
            rdma_cw.start()
            rdma_ccw.start()
            rdma_cw.wait()
            rdma_ccw.wait()
            comb_cw = (recv_cw_ref[s].astype(jnp.float32)
                       + partial_ref[pl.ds(cw_recv_idx * rows_per, half), :])
            comb_ccw = (recv_ccw_ref[s].astype(jnp.float32)
                        + partial_ref[pl.ds(ccw_recv_idx * rows_per + half, half), :])
            if s < n_steps - 1:
                send_cw_ref[:, :] = comb_cw.astype(jnp.bfloat16)
                send_ccw_ref[:, :] = comb_ccw.astype(jnp.bfloat16)
            else:
                out_ref[0:half, :] = comb_cw
                out_ref[half:rows_per, :] = comb_ccw

    return pl.pallas_call(
        body,
        out_shape=jax.ShapeDtypeStruct((rows_per, d_hidden), jnp.float32),
        in_specs=[
            pl.BlockSpec(memory_space=pltpu.VMEM),
            pl.BlockSpec(memory_space=pltpu.VMEM),
            pl.BlockSpec(memory_space=pltpu.VMEM),
            pl.BlockSpec(memory_space=pltpu.VMEM),
        ],
        out_specs=pl.BlockSpec(memory_space=pltpu.VMEM),
        scratch_shapes=[
            pltpu.VMEM((n_tok, d_hidden), jnp.float32),
            pltpu.VMEM((half, d_hidden), jnp.bfloat16),
            pltpu.VMEM((half, d_hidden), jnp.bfloat16),
            pltpu.VMEM((n_steps, half, d_hidden), jnp.bfloat16),
            pltpu.VMEM((n_steps, half, d_hidden), jnp.bfloat16),
            pltpu.SemaphoreType.DMA((n_steps,)),
            pltpu.SemaphoreType.DMA((n_steps,)),
            pltpu.SemaphoreType.DMA((n_steps,)),
            pltpu.SemaphoreType.DMA((n_steps,)),
        ],
        compiler_params=pltpu.CompilerParams(collective_id=0),
    )(x, router_W, route_idx, expert_W)


# device time: 55656 ns/iter; 2.2317x vs baseline; 1.2230x over previous
import jax
import jax.numpy as jnp
from jax import lax
from jax.experimental import pallas as pl
from jax.experimental.pallas import tpu as pltpu

N_DEV = 8


def kernel(x, router_W, route_idx, expert_W):
    n_tok, d_model = x.shape
    n_local, _, d_hidden = expert_W.shape
    rows_per = n_tok // N_DEV
    n_experts = router_W.shape[1]
    n_steps = N_DEV - 1

    half = rows_per // 2

    def body(x_ref, rw_ref, idx_ref, ew_ref, out_ref,
             partial_ref, wt_ref, send_cw_ref, send_ccw_ref,
             recv_cw_ref, recv_ccw_ref,
             send_cw_sems, recv_cw_sems, send_ccw_sems, recv_ccw_sems):
        my = lax.axis_index("i")
        left = lax.rem(my + N_DEV - 1, N_DEV)
        right = lax.rem(my + 1, N_DEV)

        xf = x_ref[:, :]
        scores = jnp.dot(xf, rw_ref[:, :], preferred_element_type=jnp.float32)
        m = jnp.max(scores, axis=1, keepdims=True)
        p = jnp.exp(scores - m)
        denom = jnp.sum(p, axis=1, keepdims=True)
        probs = p / denom

        cols = lax.broadcasted_iota(jnp.int32, (n_tok, n_experts), 1)
        idx0 = idx_ref[:, 0:1]
        idx1 = idx_ref[:, 1:2]
        g0 = jnp.sum(jnp.where(cols == idx0, probs, 0.0), axis=1, keepdims=True)
        g1 = jnp.sum(jnp.where(cols == idx1, probs, 0.0), axis=1, keepdims=True)
        gs = g0 + g1
        w0 = g0 / gs
        w1 = g1 / gs
        for e in range(n_local):
            ge = my * n_local + e
            wt_ref[:, e:e + 1] = (jnp.where(idx0 == ge, w0, 0.0)
                                  + jnp.where(idx1 == ge, w1, 0.0))

        def compute_chunk(c):
            row0 = c * rows_per
            xc = x_ref[pl.ds(row0, rows_per), :]
            acc = jnp.zeros((rows_per, d_hidden), dtype=jnp.float32)
            for e in range(n_local):
                wtc = wt_ref[pl.ds(row0, rows_per), e:e + 1]
                xs = (xc * wtc).astype(jnp.bfloat16)
                we = ew_ref[e].astype(jnp.bfloat16)
                acc = acc + jnp.dot(xs, we, preferred_element_type=jnp.float32)
            partial_ref[pl.ds(row0, rows_per), :] = acc

        compute_chunk(lax.rem(my + n_steps, N_DEV))
        compute_chunk(lax.rem(my + 1, N_DEV))

        barrier_sem = pltpu.get_barrier_semaphore()
        for nbr in (left, right):
            pl.semaphore_signal(
                barrier_sem, inc=1,
                device_id=(nbr,), device_id_type=pl.DeviceIdType.MESH,
            )
        pl.semaphore_wait(barrier_sem, 2)

        overlap_sched = {
            0: (n_steps - 1, 2),
            1: (n_steps - 2, 3),
            2: (n_steps - 3, 0),
        }
        for s in range(n_steps):
            cw_send_idx = lax.rem(my + (n_steps - s), N_DEV)
            cw_recv_idx = lax.rem(my + (n_steps - s - 1), N_DEV)
            ccw_send_idx = lax.rem(my + s + 1, N_DEV)
            ccw_recv_idx = lax.rem(my + s + 2, N_DEV)
            if s == 0:
                send_cw_ref[:, :] = partial_ref[
                    pl.ds(cw_send_idx * rows_per, half), :
                ].astype(jnp.bfloat16)
                send_ccw_ref[:, :] = partial_ref[
                    pl.ds(ccw_send_idx * rows_per + half, half), :
                ].astype(jnp.bfloat16)
            rdma_cw = pltpu.make_async_remote_copy(
                src_ref=send_cw_ref,
                dst_ref=recv_cw_ref.at[s],
                send_sem=send_cw_sems.at[s],
                recv_sem=recv_cw_sems.at[s],
                device_id=(right,),
                device_id_type=pl.DeviceIdType.MESH,
            )
            rdma_ccw = pltpu.make_async_remote_copy(
                src_ref=send_ccw_ref,
                dst_ref=recv_ccw_ref.at[s],
                send_sem=send_ccw_sems.at[s],
                recv_sem=recv_ccw_sems.at[s],
                device_id=(left,),
                device_id_type=pl.DeviceIdType.MESH,
            )
            rdma_cw.start()
            rdma_ccw.start()
            if s in overlap_sched:
                for off in overlap_sched[s]:
                    compute_chunk(lax.rem(my + off, N_DEV))
            rdma_cw.wait()
            rdma_ccw.wait()
            comb_cw = (recv_cw_ref[s].astype(jnp.float32)
                       + partial_ref[pl.ds(cw_recv_idx * rows_per, half), :])
            comb_ccw = (recv_ccw_ref[s].astype(jnp.float32)
                        + partial_ref[pl.ds(ccw_recv_idx * rows_per + half, half), :])
            if s < n_steps - 1:
                send_cw_ref[:, :] = comb_cw.astype(jnp.bfloat16)
                send_ccw_ref[:, :] = comb_ccw.astype(jnp.bfloat16)
            else:
                out_ref[0:half, :] = comb_cw
                out_ref[half:rows_per, :] = comb_ccw

    return pl.pallas_call(
        body,
        out_shape=jax.ShapeDtypeStruct((rows_per, d_hidden), jnp.float32),
        in_specs=[
            pl.BlockSpec(memory_space=pltpu.VMEM),
            pl.BlockSpec(memory_space=pltpu.VMEM),
            pl.BlockSpec(memory_space=pltpu.VMEM),
            pl.BlockSpec(memory_space=pltpu.VMEM),
        ],
        out_specs=pl.BlockSpec(memory_space=pltpu.VMEM),
        scratch_shapes=[
            pltpu.VMEM((n_tok, d_hidden), jnp.float32),
            pltpu.VMEM((n_tok, n_local), jnp.float32),
            pltpu.VMEM((half, d_hidden), jnp.bfloat16),
            pltpu.VMEM((half, d_hidden), jnp.bfloat16),
            pltpu.VMEM((n_steps, half, d_hidden), jnp.bfloat16),
            pltpu.VMEM((n_steps, half, d_hidden), jnp.bfloat16),
            pltpu.SemaphoreType.DMA((n_steps,)),
            pltpu.SemaphoreType.DMA((n_steps,)),
            pltpu.SemaphoreType.DMA((n_steps,)),
            pltpu.SemaphoreType.DMA((n_steps,)),
        ],
        compiler_params=pltpu.CompilerParams(collective_id=0),
    )(x, router_W, route_idx, expert_W)


# device time: 55030 ns/iter; 2.2571x vs baseline; 1.0114x over previous
import jax
import jax.numpy as jnp
from jax import lax
from jax.experimental import pallas as pl
from jax.experimental.pallas import tpu as pltpu

N_DEV = 8


def kernel(x, router_W, route_idx, expert_W):
    n_tok, d_model = x.shape
    n_local, _, d_hidden = expert_W.shape
    rows_per = n_tok // N_DEV
    n_experts = router_W.shape[1]
    n_steps = N_DEV - 1

    half = rows_per // 2

    def body(x_ref, rw_ref, idx_ref, ew_ref, out_ref,
             wt_ref, send_cw_ref, send_ccw_ref,
             recv_cw_ref, recv_ccw_ref,
             send_cw_sems, recv_cw_sems, send_ccw_sems, recv_ccw_sems):
        my = lax.axis_index("i")
        left = lax.rem(my + N_DEV - 1, N_DEV)
        right = lax.rem(my + 1, N_DEV)

        xf = x_ref[:, :]
        scores = jnp.dot(xf, rw_ref[:, :], preferred_element_type=jnp.float32)
        m = jnp.max(scores, axis=1, keepdims=True)
        p = jnp.exp(scores - m)
        denom = jnp.sum(p, axis=1, keepdims=True)
        probs = p / denom

        cols = lax.broadcasted_iota(jnp.int32, (n_tok, n_experts), 1)
        idx0 = idx_ref[:, 0:1]
        idx1 = idx_ref[:, 1:2]
        g0 = jnp.sum(jnp.where(cols == idx0, probs, 0.0), axis=1, keepdims=True)
        g1 = jnp.sum(jnp.where(cols == idx1, probs, 0.0), axis=1, keepdims=True)
        gs = g0 + g1
        w0 = g0 / gs
        w1 = g1 / gs
        for e in range(n_local):
            ge = my * n_local + e
            wt_ref[:, e:e + 1] = (jnp.where(idx0 == ge, w0, 0.0)
                                  + jnp.where(idx1 == ge, w1, 0.0))

        def compute_half(c, top):
            row0 = c * rows_per + (0 if top else half)
            xc = x_ref[pl.ds(row0, half), :]
            acc = jnp.zeros((half, d_hidden), dtype=jnp.float32)
            for e in range(n_local):
                wtc = wt_ref[pl.ds(row0, half), e:e + 1]
                xs = (xc * wtc).astype(jnp.bfloat16)
                we = ew_ref[e].astype(jnp.bfloat16)
                acc = acc + jnp.dot(xs, we, preferred_element_type=jnp.float32)
            return acc

        send_cw_ref[:, :] = compute_half(
            lax.rem(my + n_steps, N_DEV), True).astype(jnp.bfloat16)
        send_ccw_ref[:, :] = compute_half(
            lax.rem(my + 1, N_DEV), False).astype(jnp.bfloat16)

        barrier_sem = pltpu.get_barrier_semaphore()
        for nbr in (left, right):
            pl.semaphore_signal(
                barrier_sem, inc=1,
                device_id=(nbr,), device_id_type=pl.DeviceIdType.MESH,
            )
        pl.semaphore_wait(barrier_sem, 2)

        for s in range(n_steps):
            cw_recv_idx = lax.rem(my + (n_steps - s - 1), N_DEV)
            ccw_recv_idx = lax.rem(my + s + 2, N_DEV)
            rdma_cw = pltpu.make_async_remote_copy(
                src_ref=send_cw_ref,
                dst_ref=recv_cw_ref.at[s],
                send_sem=send_cw_sems.at[s],
                recv_sem=recv_cw_sems.at[s],
                device_id=(right,),
                device_id_type=pl.DeviceIdType.MESH,
            )
            rdma_ccw = pltpu.make_async_remote_copy(
                src_ref=send_ccw_ref,
                dst_ref=recv_ccw_ref.at[s],
                send_sem=send_ccw_sems.at[s],
                recv_sem=recv_ccw_sems.at[s],
                device_id=(left,),
                device_id_type=pl.DeviceIdType.MESH,
            )
            rdma_cw.start()
            rdma_ccw.start()
            acc_cw = compute_half(cw_recv_idx, True)
            acc_ccw = compute_half(ccw_recv_idx, False)
            rdma_cw.wait()
            rdma_ccw.wait()
            comb_cw = recv_cw_ref[s].astype(jnp.float32) + acc_cw
            comb_ccw = recv_ccw_ref[s].astype(jnp.float32) + acc_ccw
            if s < n_steps - 1:
                send_cw_ref[:, :] = comb_cw.astype(jnp.bfloat16)
                send_ccw_ref[:, :] = comb_ccw.astype(jnp.bfloat16)
            else:
                out_ref[0:half, :] = comb_cw
                out_ref[half:rows_per, :] = comb_ccw

    return pl.pallas_call(
        body,
        out_shape=jax.ShapeDtypeStruct((rows_per, d_hidden), jnp.float32),
        in_specs=[
            pl.BlockSpec(memory_space=pltpu.VMEM),
            pl.BlockSpec(memory_space=pltpu.VMEM),
            pl.BlockSpec(memory_space=pltpu.VMEM),
            pl.BlockSpec(memory_space=pltpu.VMEM),
        ],
        out_specs=pl.BlockSpec(memory_space=pltpu.VMEM),
        scratch_shapes=[
            pltpu.VMEM((n_tok, n_local), jnp.float32),
            pltpu.VMEM((half, d_hidden), jnp.bfloat16),
            pltpu.VMEM((half, d_hidden), jnp.bfloat16),
            pltpu.VMEM((n_steps, half, d_hidden), jnp.bfloat16),
            pltpu.VMEM((n_steps, half, d_hidden), jnp.bfloat16),
            pltpu.SemaphoreType.DMA((n_steps,)),
            pltpu.SemaphoreType.DMA((n_steps,)),
            pltpu.SemaphoreType.DMA((n_steps,)),
            pltpu.SemaphoreType.DMA((n_steps,)),
        ],
        compiler_params=pltpu.CompilerParams(collective_id=0),
    )(x, router_W, route_idx, expert_W)
